# Initial kernel scaffold; baseline (speedup 1.0000x reference)
import jax
import jax.numpy as jnp
from jax.experimental import pallas as pl


def kernel(xyz_t, feat_t, xyz_t1, feat_t1, W1, W2):
    raise NotImplementedError("write your pallas kernel here")


# speedup vs baseline: 15.8010x; 15.8010x over previous
import functools

import jax
import jax.numpy as jnp
import numpy as np
from jax import lax
from jax.experimental import pallas as pl
from jax.experimental.pallas import tpu as pltpu
from jax.experimental.pallas import tpu_sc as plsc

_B, _N, _K = 2, 4096, 16
_CT, _CT1, _OUT = 64, 64, 128
_H = _OUT // 2
_CIN = _CT + _CT1 + 3
_R2 = np.float32(0.1 * 0.1)
_EPS = np.float32(1e-5)
_TOT = _B * _N * _K

_QT = 256


def _ballq_body(q_ref, bT_ref, idx_ref):
    b = pl.program_id(0)
    q = q_ref[0]
    sqd = None
    for d in range(3):
        diff = q[:, d : d + 1] - bT_ref[0, d : d + 1, :]
        sq = diff * diff
        sqd = sq if sqd is None else sqd + sq
    iota = lax.broadcasted_iota(jnp.int32, (_QT, _N), 1)
    key = jnp.where(sqd < _R2, iota, _N)
    off = b * _N
    m = jnp.min(key, axis=1, keepdims=True)
    firstfix = jnp.where(m == _N, 0, m) + off
    idx_ref[0, :, 0:1] = firstfix
    for j in range(1, _K):
        key = jnp.where(key == m, _N, key)
        m = jnp.min(key, axis=1, keepdims=True)
        idx_ref[0, :, j : j + 1] = jnp.where(m == _N, firstfix, m + off)


def _prep_body(ft1_ref, x1_ref, ft_ref, x_ref, w1t_ref, t_ref, qt_ref):
    w1bt = w1t_ref[_CT : _CT + _CT1, :]
    t = jnp.dot(ft1_ref[...], w1bt, preferred_element_type=jnp.float32,
                precision=lax.Precision.HIGHEST)
    qt = jnp.dot(ft_ref[...], w1t_ref[0:_CT, :], preferred_element_type=jnp.float32,
                 precision=lax.Precision.HIGHEST)
    for d in range(3):
        wrow = w1t_ref[_CT + _CT1 + d : _CT + _CT1 + d + 1, :]
        t = t + x1_ref[:, d : d + 1] * wrow
        qt = qt - x_ref[:, d : d + 1] * wrow
    t_ref[:, 0:_H] = t
    t_ref[:, _H:] = jnp.zeros_like(t)
    qt_ref[...] = qt


def _stats1_body(g_ref, qt_ref, s1_ref, acc_ref):
    step = pl.program_id(0) * _K + pl.program_id(1)
    y = g_ref[0][:, 0:_H] + qt_ref[0]

    @pl.when(step == 0)
    def _():
        acc_ref[...] = jnp.zeros_like(acc_ref)

    acc_ref[0:1, :] += jnp.sum(y, axis=0, keepdims=True)
    acc_ref[1:2, :] += jnp.sum(y * y, axis=0, keepdims=True)

    @pl.when(step == _B * _K - 1)
    def _():
        s1_ref[...] = acc_ref[...]


def _mlp_body(g_ref, qt_ref, s1_ref, w2t_ref, zmax_ref, s2_ref, acc_ref):
    b = pl.program_id(0)
    k = pl.program_id(1)
    step = b * _K + k
    cnt = np.float32(_TOT)
    mu = s1_ref[0:1, :] / cnt
    var = s1_ref[1:2, :] / cnt - mu * mu
    rsig = lax.rsqrt(var + _EPS)
    y = (g_ref[0][:, 0:_H] + qt_ref[0] - mu) * rsig
    y = jnp.maximum(y, 0.0)
    z = jnp.dot(y, w2t_ref[...], preferred_element_type=jnp.float32,
                precision=lax.Precision.HIGHEST)

    @pl.when(step == 0)
    def _():
        acc_ref[...] = jnp.zeros_like(acc_ref)

    acc_ref[0:1, :] += jnp.sum(z, axis=0, keepdims=True)
    acc_ref[1:2, :] += jnp.sum(z * z, axis=0, keepdims=True)

    @pl.when(k == 0)
    def _():
        zmax_ref[0] = z

    @pl.when(k > 0)
    def _():
        zmax_ref[0] = jnp.maximum(zmax_ref[0], z)

    @pl.when(step == _B * _K - 1)
    def _():
        s2_ref[...] = acc_ref[...]


def _final_body(zmax_ref, s2_ref, out_ref):
    cnt = np.float32(_TOT)
    mu = s2_ref[0:1, :] / cnt
    var = s2_ref[1:2, :] / cnt - mu * mu
    rsig = lax.rsqrt(var + _EPS)
    out_ref[0] = jnp.maximum((zmax_ref[0] - mu) * rsig, 0.0)


def _gather_rows(t, idx2d):
    nw = 32
    nrow128 = _TOT // 128
    cpw = nrow128 // nw
    grp = 4
    mesh = plsc.VectorSubcoreMesh(core_axis_name="c", subcore_axis_name="s")

    @functools.partial(
        pl.kernel,
        out_type=jax.ShapeDtypeStruct((_TOT, 128), jnp.float32),
        mesh=mesh,
        scratch_types=[
            pltpu.VMEM((cpw, 128), jnp.int32),
            pltpu.VMEM((grp * 128, 128), jnp.float32),
            pltpu.SemaphoreType.DMA,
        ],
    )
    def gk(idx_hbm, t_hbm, out_hbm, idx_v, rows_v, sem):
        wid = lax.axis_index("s") * 2 + lax.axis_index("c")
        cbase = wid * cpw
        pltpu.sync_copy(idx_hbm.at[pl.ds(cbase, cpw)], idx_v)
        for gi in range(cpw // grp):
            cps = [
                pltpu.async_copy(
                    t_hbm.at[idx_v.at[gi * grp + j]],
                    rows_v.at[pl.ds(j * 128, 128)],
                    sem,
                )
                for j in range(grp)
            ]
            for cp in cps:
                cp.wait()
            pltpu.sync_copy(
                rows_v, out_hbm.at[pl.ds((cbase + gi * grp) * 128, grp * 128)]
            )

    return gk(idx2d, t)


def kernel(xyz_t, feat_t, xyz_t1, feat_t1, W1, W2):
    bT = jnp.transpose(xyz_t1, (0, 2, 1))
    idx = pl.pallas_call(
        _ballq_body,
        grid=(_B, _N // _QT),
        in_specs=[
            pl.BlockSpec((1, _QT, 3), lambda b, i: (b, i, 0)),
            pl.BlockSpec((1, 3, _N), lambda b, i: (b, 0, 0)),
        ],
        out_specs=pl.BlockSpec((1, _QT, _K), lambda b, i: (b, i, 0)),
        out_shape=jax.ShapeDtypeStruct((_B, _N, _K), jnp.int32),
    )(xyz_t, bT)

    W1T = W1.T
    rt = 1024
    t, qterm = pl.pallas_call(
        _prep_body,
        grid=(_B * _N // rt,),
        in_specs=[
            pl.BlockSpec((rt, _CT1), lambda i: (i, 0)),
            pl.BlockSpec((rt, 3), lambda i: (i, 0)),
            pl.BlockSpec((rt, _CT), lambda i: (i, 0)),
            pl.BlockSpec((rt, 3), lambda i: (i, 0)),
            pl.BlockSpec((_CIN, _H), lambda i: (0, 0)),
        ],
        out_specs=[
            pl.BlockSpec((rt, 128), lambda i: (i, 0)),
            pl.BlockSpec((rt, _H), lambda i: (i, 0)),
        ],
        out_shape=[
            jax.ShapeDtypeStruct((_B * _N, 128), jnp.float32),
            jax.ShapeDtypeStruct((_B * _N, _H), jnp.float32),
        ],
    )(
        feat_t1.reshape(_B * _N, _CT1),
        xyz_t1.reshape(_B * _N, 3),
        feat_t.reshape(_B * _N, _CT),
        xyz_t.reshape(_B * _N, 3),
        W1T,
    )

    idx2d = jnp.transpose(idx, (0, 2, 1)).reshape(_TOT // 128, 128)
    g = _gather_rows(t, idx2d)

    g3 = g.reshape(_B * _K, _N, 128)
    qt3 = qterm.reshape(_B, _N, _H)

    s1 = pl.pallas_call(
        _stats1_body,
        grid=(_B, _K),
        in_specs=[
            pl.BlockSpec((1, _N, 128), lambda b, k: (b * _K + k, 0, 0)),
            pl.BlockSpec((1, _N, _H), lambda b, k: (b, 0, 0)),
        ],
        out_specs=pl.BlockSpec((8, _H), lambda b, k: (0, 0)),
        out_shape=jax.ShapeDtypeStruct((8, _H), jnp.float32),
        scratch_shapes=[pltpu.VMEM((8, _H), jnp.float32)],
    )(g3, qt3)

    zmax, s2 = pl.pallas_call(
        _mlp_body,
        grid=(_B, _K),
        in_specs=[
            pl.BlockSpec((1, _N, 128), lambda b, k: (b * _K + k, 0, 0)),
            pl.BlockSpec((1, _N, _H), lambda b, k: (b, 0, 0)),
            pl.BlockSpec((8, _H), lambda b, k: (0, 0)),
            pl.BlockSpec((_H, _OUT), lambda b, k: (0, 0)),
        ],
        out_specs=[
            pl.BlockSpec((1, _N, _OUT), lambda b, k: (b, 0, 0)),
            pl.BlockSpec((8, _OUT), lambda b, k: (0, 0)),
        ],
        out_shape=[
            jax.ShapeDtypeStruct((_B, _N, _OUT), jnp.float32),
            jax.ShapeDtypeStruct((8, _OUT), jnp.float32),
        ],
        scratch_shapes=[pltpu.VMEM((8, _OUT), jnp.float32)],
    )(g3, qt3, s1, W2.T)

    out = pl.pallas_call(
        _final_body,
        grid=(_B,),
        in_specs=[
            pl.BlockSpec((1, _N, _OUT), lambda b: (b, 0, 0)),
            pl.BlockSpec((8, _OUT), lambda b: (0, 0)),
        ],
        out_specs=pl.BlockSpec((1, _N, _OUT), lambda b: (b, 0, 0)),
        out_shape=jax.ShapeDtypeStruct((_B, _N, _OUT), jnp.float32),
    )(zmax, s2)
    return out


# speedup vs baseline: 15.8353x; 1.0022x over previous
import functools

import jax
import jax.numpy as jnp
import numpy as np
from jax import lax
from jax.experimental import pallas as pl
from jax.experimental.pallas import tpu as pltpu
from jax.experimental.pallas import tpu_sc as plsc

_B, _N, _K = 2, 4096, 16
_CT, _CT1, _OUT = 64, 64, 128
_H = _OUT // 2
_CIN = _CT + _CT1 + 3
_R2 = np.float32(0.1 * 0.1)
_EPS = np.float32(1e-5)
_TOT = _B * _N * _K

_QT = 256


def _ballq_body(q_ref, bT_ref, idx_ref):
    b = pl.program_id(0)
    q = q_ref[0]
    sqd = None
    for d in range(3):
        diff = q[:, d : d + 1] - bT_ref[0, d : d + 1, :]
        sq = diff * diff
        sqd = sq if sqd is None else sqd + sq
    iota = lax.broadcasted_iota(jnp.int32, (_QT, _N), 1)
    key = jnp.where(sqd < _R2, iota, _N)
    off = b * _N
    m = jnp.min(key, axis=1, keepdims=True)
    firstfix = jnp.where(m == _N, 0, m) + off
    idx_ref[0, :, 0:1] = firstfix
    for j in range(1, _K):
        m = jnp.min(jnp.where(key > m, key, _N), axis=1, keepdims=True)
        idx_ref[0, :, j : j + 1] = jnp.where(m == _N, firstfix, m + off)


def _prep_body(ft1_ref, x1_ref, ft_ref, x_ref, w1t_ref, t_ref, qt_ref):
    w1bt = w1t_ref[_CT : _CT + _CT1, :]
    t = jnp.dot(ft1_ref[...], w1bt, preferred_element_type=jnp.float32,
                precision=lax.Precision.HIGHEST)
    qt = jnp.dot(ft_ref[...], w1t_ref[0:_CT, :], preferred_element_type=jnp.float32,
                 precision=lax.Precision.HIGHEST)
    for d in range(3):
        wrow = w1t_ref[_CT + _CT1 + d : _CT + _CT1 + d + 1, :]
        t = t + x1_ref[:, d : d + 1] * wrow
        qt = qt - x_ref[:, d : d + 1] * wrow
    t_ref[:, 0:_H] = t
    t_ref[:, _H:] = jnp.zeros_like(t)
    qt_ref[...] = qt


def _stats1_body(g_ref, qt_ref, s1_ref, acc_ref):
    step = pl.program_id(0) * _K + pl.program_id(1)
    y = g_ref[0][:, 0:_H] + qt_ref[0]

    @pl.when(step == 0)
    def _():
        acc_ref[...] = jnp.zeros_like(acc_ref)

    acc_ref[0:1, :] += jnp.sum(y, axis=0, keepdims=True)
    acc_ref[1:2, :] += jnp.sum(y * y, axis=0, keepdims=True)

    @pl.when(step == _B * _K - 1)
    def _():
        s1_ref[...] = acc_ref[...]


def _mlp_body(g_ref, qt_ref, s1_ref, w2t_ref, zmax_ref, s2_ref, acc_ref):
    b = pl.program_id(0)
    k = pl.program_id(1)
    step = b * _K + k
    cnt = np.float32(_TOT)
    mu = s1_ref[0:1, :] / cnt
    var = s1_ref[1:2, :] / cnt - mu * mu
    rsig = lax.rsqrt(var + _EPS)
    y = (g_ref[0][:, 0:_H] + qt_ref[0] - mu) * rsig
    y = jnp.maximum(y, 0.0)
    z = jnp.dot(y, w2t_ref[...], preferred_element_type=jnp.float32,
                precision=lax.Precision.HIGHEST)

    @pl.when(step == 0)
    def _():
        acc_ref[...] = jnp.zeros_like(acc_ref)

    acc_ref[0:1, :] += jnp.sum(z, axis=0, keepdims=True)
    acc_ref[1:2, :] += jnp.sum(z * z, axis=0, keepdims=True)

    @pl.when(k == 0)
    def _():
        zmax_ref[0] = z

    @pl.when(k > 0)
    def _():
        zmax_ref[0] = jnp.maximum(zmax_ref[0], z)

    @pl.when(step == _B * _K - 1)
    def _():
        s2_ref[...] = acc_ref[...]


def _final_body(zmax_ref, s2_ref, out_ref):
    cnt = np.float32(_TOT)
    mu = s2_ref[0:1, :] / cnt
    var = s2_ref[1:2, :] / cnt - mu * mu
    rsig = lax.rsqrt(var + _EPS)
    out_ref[0] = jnp.maximum((zmax_ref[0] - mu) * rsig, 0.0)


def _gather_rows(t, idx2d):
    nw = 32
    nrow128 = _TOT // 128
    cpw = nrow128 // nw
    grp = 4
    mesh = plsc.VectorSubcoreMesh(core_axis_name="c", subcore_axis_name="s")

    @functools.partial(
        pl.kernel,
        out_type=jax.ShapeDtypeStruct((_TOT, 128), jnp.float32),
        mesh=mesh,
        scratch_types=[
            pltpu.VMEM((cpw, 128), jnp.int32),
            pltpu.VMEM((grp * 128, 128), jnp.float32),
            pltpu.SemaphoreType.DMA,
        ],
    )
    def gk(idx_hbm, t_hbm, out_hbm, idx_v, rows_v, sem):
        wid = lax.axis_index("s") * 2 + lax.axis_index("c")
        cbase = wid * cpw
        pltpu.sync_copy(idx_hbm.at[pl.ds(cbase, cpw)], idx_v)
        for gi in range(cpw // grp):
            cps = [
                pltpu.async_copy(
                    t_hbm.at[idx_v.at[gi * grp + j]],
                    rows_v.at[pl.ds(j * 128, 128)],
                    sem,
                )
                for j in range(grp)
            ]
            for cp in cps:
                cp.wait()
            pltpu.sync_copy(
                rows_v, out_hbm.at[pl.ds((cbase + gi * grp) * 128, grp * 128)]
            )

    return gk(idx2d, t)


def kernel(xyz_t, feat_t, xyz_t1, feat_t1, W1, W2):
    bT = jnp.transpose(xyz_t1, (0, 2, 1))
    idx = pl.pallas_call(
        _ballq_body,
        grid=(_B, _N // _QT),
        in_specs=[
            pl.BlockSpec((1, _QT, 3), lambda b, i: (b, i, 0)),
            pl.BlockSpec((1, 3, _N), lambda b, i: (b, 0, 0)),
        ],
        out_specs=pl.BlockSpec((1, _QT, _K), lambda b, i: (b, i, 0)),
        out_shape=jax.ShapeDtypeStruct((_B, _N, _K), jnp.int32),
    )(xyz_t, bT)

    W1T = W1.T
    rt = 1024
    t, qterm = pl.pallas_call(
        _prep_body,
        grid=(_B * _N // rt,),
        in_specs=[
            pl.BlockSpec((rt, _CT1), lambda i: (i, 0)),
            pl.BlockSpec((rt, 3), lambda i: (i, 0)),
            pl.BlockSpec((rt, _CT), lambda i: (i, 0)),
            pl.BlockSpec((rt, 3), lambda i: (i, 0)),
            pl.BlockSpec((_CIN, _H), lambda i: (0, 0)),
        ],
        out_specs=[
            pl.BlockSpec((rt, 128), lambda i: (i, 0)),
            pl.BlockSpec((rt, _H), lambda i: (i, 0)),
        ],
        out_shape=[
            jax.ShapeDtypeStruct((_B * _N, 128), jnp.float32),
            jax.ShapeDtypeStruct((_B * _N, _H), jnp.float32),
        ],
    )(
        feat_t1.reshape(_B * _N, _CT1),
        xyz_t1.reshape(_B * _N, 3),
        feat_t.reshape(_B * _N, _CT),
        xyz_t.reshape(_B * _N, 3),
        W1T,
    )

    idx2d = jnp.transpose(idx, (0, 2, 1)).reshape(_TOT // 128, 128)
    g = _gather_rows(t, idx2d)

    g3 = g.reshape(_B * _K, _N, 128)
    qt3 = qterm.reshape(_B, _N, _H)

    s1 = pl.pallas_call(
        _stats1_body,
        grid=(_B, _K),
        in_specs=[
            pl.BlockSpec((1, _N, 128), lambda b, k: (b * _K + k, 0, 0)),
            pl.BlockSpec((1, _N, _H), lambda b, k: (b, 0, 0)),
        ],
        out_specs=pl.BlockSpec((8, _H), lambda b, k: (0, 0)),
        out_shape=jax.ShapeDtypeStruct((8, _H), jnp.float32),
        scratch_shapes=[pltpu.VMEM((8, _H), jnp.float32)],
    )(g3, qt3)

    zmax, s2 = pl.pallas_call(
        _mlp_body,
        grid=(_B, _K),
        in_specs=[
            pl.BlockSpec((1, _N, 128), lambda b, k: (b * _K + k, 0, 0)),
            pl.BlockSpec((1, _N, _H), lambda b, k: (b, 0, 0)),
            pl.BlockSpec((8, _H), lambda b, k: (0, 0)),
            pl.BlockSpec((_H, _OUT), lambda b, k: (0, 0)),
        ],
        out_specs=[
            pl.BlockSpec((1, _N, _OUT), lambda b, k: (b, 0, 0)),
            pl.BlockSpec((8, _OUT), lambda b, k: (0, 0)),
        ],
        out_shape=[
            jax.ShapeDtypeStruct((_B, _N, _OUT), jnp.float32),
            jax.ShapeDtypeStruct((8, _OUT), jnp.float32),
        ],
        scratch_shapes=[pltpu.VMEM((8, _OUT), jnp.float32)],
    )(g3, qt3, s1, W2.T)

    out = pl.pallas_call(
        _final_body,
        grid=(_B,),
        in_specs=[
            pl.BlockSpec((1, _N, _OUT), lambda b: (b, 0, 0)),
            pl.BlockSpec((8, _OUT), lambda b: (0, 0)),
        ],
        out_specs=pl.BlockSpec((1, _N, _OUT), lambda b: (b, 0, 0)),
        out_shape=jax.ShapeDtypeStruct((_B, _N, _OUT), jnp.float32),
    )(zmax, s2)
    return out


# speedup vs baseline: 18.3665x; 1.1598x over previous
import functools

import jax
import jax.numpy as jnp
import numpy as np
from jax import lax
from jax.experimental import pallas as pl
from jax.experimental.pallas import tpu as pltpu
from jax.experimental.pallas import tpu_sc as plsc

_B, _N, _K = 2, 4096, 16
_CT, _CT1, _OUT = 64, 64, 128
_H = _OUT // 2
_CIN = _CT + _CT1 + 3
_R2 = np.float32(0.1 * 0.1)
_EPS = np.float32(1e-5)
_TOT = _B * _N * _K

_QT = 256


def _ballq_body(q_ref, bT_ref, idx_ref):
    b = pl.program_id(0)
    q = q_ref[0]
    sqd = None
    for d in range(3):
        diff = q[:, d : d + 1] - bT_ref[0, d : d + 1, :]
        sq = diff * diff
        sqd = sq if sqd is None else sqd + sq
    iota = lax.broadcasted_iota(jnp.int32, (_QT, _N), 1)
    key = jnp.where(sqd < _R2, iota, _N)
    off = b * _N
    m = jnp.min(key, axis=1, keepdims=True)
    firstfix = jnp.where(m == _N, 0, m) + off
    idx_ref[0, :, 0:1] = firstfix
    imin = jnp.int32(-(2**31))
    keym = key + imin
    for j in range(1, _K):
        dm = jnp.min(keym - (m + 1), axis=1, keepdims=True)
        m = jnp.where(dm < 0, m + 1 + (dm ^ imin), _N)
        idx_ref[0, :, j : j + 1] = jnp.where(m == _N, firstfix, m + off)


def _prep_body(ft1_ref, x1_ref, ft_ref, x_ref, w1t_ref, t_ref, qt_ref):
    w1bt = w1t_ref[_CT : _CT + _CT1, :]
    t = jnp.dot(ft1_ref[...], w1bt, preferred_element_type=jnp.float32,
                precision=lax.Precision.HIGHEST)
    qt = jnp.dot(ft_ref[...], w1t_ref[0:_CT, :], preferred_element_type=jnp.float32,
                 precision=lax.Precision.HIGHEST)
    for d in range(3):
        wrow = w1t_ref[_CT + _CT1 + d : _CT + _CT1 + d + 1, :]
        t = t + x1_ref[:, d : d + 1] * wrow
        qt = qt - x_ref[:, d : d + 1] * wrow
    t_ref[:, 0:_H] = t
    t_ref[:, _H:] = jnp.zeros_like(t)
    qt_ref[...] = qt


def _stats1_body(g_ref, qt_ref, s1_ref, acc_ref):
    step = pl.program_id(0) * _K + pl.program_id(1)
    y = g_ref[0][:, 0:_H] + qt_ref[0]

    @pl.when(step == 0)
    def _():
        acc_ref[...] = jnp.zeros_like(acc_ref)

    acc_ref[0:1, :] += jnp.sum(y, axis=0, keepdims=True)
    acc_ref[1:2, :] += jnp.sum(y * y, axis=0, keepdims=True)

    @pl.when(step == _B * _K - 1)
    def _():
        s1_ref[...] = acc_ref[...]


def _mlp_body(g_ref, qt_ref, s1_ref, w2t_ref, zmax_ref, s2_ref, acc_ref):
    b = pl.program_id(0)
    k = pl.program_id(1)
    step = b * _K + k
    cnt = np.float32(_TOT)
    mu = s1_ref[0:1, :] / cnt
    var = s1_ref[1:2, :] / cnt - mu * mu
    rsig = lax.rsqrt(var + _EPS)
    y = (g_ref[0][:, 0:_H] + qt_ref[0] - mu) * rsig
    y = jnp.maximum(y, 0.0)
    z = jnp.dot(y, w2t_ref[...], preferred_element_type=jnp.float32,
                precision=lax.Precision.DEFAULT)

    @pl.when(step == 0)
    def _():
        acc_ref[...] = jnp.zeros_like(acc_ref)

    acc_ref[0:1, :] += jnp.sum(z, axis=0, keepdims=True)
    acc_ref[1:2, :] += jnp.sum(z * z, axis=0, keepdims=True)

    @pl.when(k == 0)
    def _():
        zmax_ref[0] = z

    @pl.when(k > 0)
    def _():
        zmax_ref[0] = jnp.maximum(zmax_ref[0], z)

    @pl.when(step == _B * _K - 1)
    def _():
        s2_ref[...] = acc_ref[...]


def _final_body(zmax_ref, s2_ref, out_ref):
    cnt = np.float32(_TOT)
    mu = s2_ref[0:1, :] / cnt
    var = s2_ref[1:2, :] / cnt - mu * mu
    rsig = lax.rsqrt(var + _EPS)
    out_ref[0] = jnp.maximum((zmax_ref[0] - mu) * rsig, 0.0)


def _gather_rows(t, idx2d):
    nw = 32
    nrow128 = _TOT // 128
    cpw = nrow128 // nw
    grp = 4
    mesh = plsc.VectorSubcoreMesh(core_axis_name="c", subcore_axis_name="s")

    @functools.partial(
        pl.kernel,
        out_type=jax.ShapeDtypeStruct((_TOT, 128), jnp.float32),
        mesh=mesh,
        scratch_types=[
            pltpu.VMEM((cpw, 128), jnp.int32),
            pltpu.VMEM((grp * 128, 128), jnp.float32),
            pltpu.SemaphoreType.DMA,
        ],
    )
    def gk(idx_hbm, t_hbm, out_hbm, idx_v, rows_v, sem):
        wid = lax.axis_index("s") * 2 + lax.axis_index("c")
        cbase = wid * cpw
        pltpu.sync_copy(idx_hbm.at[pl.ds(cbase, cpw)], idx_v)
        for gi in range(cpw // grp):
            cps = [
                pltpu.async_copy(
                    t_hbm.at[idx_v.at[gi * grp + j]],
                    rows_v.at[pl.ds(j * 128, 128)],
                    sem,
                )
                for j in range(grp)
            ]
            for cp in cps:
                cp.wait()
            pltpu.sync_copy(
                rows_v, out_hbm.at[pl.ds((cbase + gi * grp) * 128, grp * 128)]
            )

    return gk(idx2d, t)


def kernel(xyz_t, feat_t, xyz_t1, feat_t1, W1, W2):
    bT = jnp.transpose(xyz_t1, (0, 2, 1))
    idx = pl.pallas_call(
        _ballq_body,
        grid=(_B, _N // _QT),
        in_specs=[
            pl.BlockSpec((1, _QT, 3), lambda b, i: (b, i, 0)),
            pl.BlockSpec((1, 3, _N), lambda b, i: (b, 0, 0)),
        ],
        out_specs=pl.BlockSpec((1, _QT, _K), lambda b, i: (b, i, 0)),
        out_shape=jax.ShapeDtypeStruct((_B, _N, _K), jnp.int32),
    )(xyz_t, bT)

    W1T = W1.T
    rt = 1024
    t, qterm = pl.pallas_call(
        _prep_body,
        grid=(_B * _N // rt,),
        in_specs=[
            pl.BlockSpec((rt, _CT1), lambda i: (i, 0)),
            pl.BlockSpec((rt, 3), lambda i: (i, 0)),
            pl.BlockSpec((rt, _CT), lambda i: (i, 0)),
            pl.BlockSpec((rt, 3), lambda i: (i, 0)),
            pl.BlockSpec((_CIN, _H), lambda i: (0, 0)),
        ],
        out_specs=[
            pl.BlockSpec((rt, 128), lambda i: (i, 0)),
            pl.BlockSpec((rt, _H), lambda i: (i, 0)),
        ],
        out_shape=[
            jax.ShapeDtypeStruct((_B * _N, 128), jnp.float32),
            jax.ShapeDtypeStruct((_B * _N, _H), jnp.float32),
        ],
    )(
        feat_t1.reshape(_B * _N, _CT1),
        xyz_t1.reshape(_B * _N, 3),
        feat_t.reshape(_B * _N, _CT),
        xyz_t.reshape(_B * _N, 3),
        W1T,
    )

    idx2d = jnp.transpose(idx, (0, 2, 1)).reshape(_TOT // 128, 128)
    g = _gather_rows(t, idx2d)

    g3 = g.reshape(_B * _K, _N, 128)
    qt3 = qterm.reshape(_B, _N, _H)

    s1 = pl.pallas_call(
        _stats1_body,
        grid=(_B, _K),
        in_specs=[
            pl.BlockSpec((1, _N, 128), lambda b, k: (b * _K + k, 0, 0)),
            pl.BlockSpec((1, _N, _H), lambda b, k: (b, 0, 0)),
        ],
        out_specs=pl.BlockSpec((8, _H), lambda b, k: (0, 0)),
        out_shape=jax.ShapeDtypeStruct((8, _H), jnp.float32),
        scratch_shapes=[pltpu.VMEM((8, _H), jnp.float32)],
    )(g3, qt3)

    zmax, s2 = pl.pallas_call(
        _mlp_body,
        grid=(_B, _K),
        in_specs=[
            pl.BlockSpec((1, _N, 128), lambda b, k: (b * _K + k, 0, 0)),
            pl.BlockSpec((1, _N, _H), lambda b, k: (b, 0, 0)),
            pl.BlockSpec((8, _H), lambda b, k: (0, 0)),
            pl.BlockSpec((_H, _OUT), lambda b, k: (0, 0)),
        ],
        out_specs=[
            pl.BlockSpec((1, _N, _OUT), lambda b, k: (b, 0, 0)),
            pl.BlockSpec((8, _OUT), lambda b, k: (0, 0)),
        ],
        out_shape=[
            jax.ShapeDtypeStruct((_B, _N, _OUT), jnp.float32),
            jax.ShapeDtypeStruct((8, _OUT), jnp.float32),
        ],
        scratch_shapes=[pltpu.VMEM((8, _OUT), jnp.float32)],
    )(g3, qt3, s1, W2.T)

    out = pl.pallas_call(
        _final_body,
        grid=(_B,),
        in_specs=[
            pl.BlockSpec((1, _N, _OUT), lambda b: (b, 0, 0)),
            pl.BlockSpec((8, _OUT), lambda b: (0, 0)),
        ],
        out_specs=pl.BlockSpec((1, _N, _OUT), lambda b: (b, 0, 0)),
        out_shape=jax.ShapeDtypeStruct((_B, _N, _OUT), jnp.float32),
    )(zmax, s2)
    return out


# speedup vs baseline: 24.0158x; 1.3076x over previous
import functools

import jax
import jax.numpy as jnp
import numpy as np
from jax import lax
from jax.experimental import pallas as pl
from jax.experimental.pallas import tpu as pltpu
from jax.experimental.pallas import tpu_sc as plsc

_B, _N, _K = 2, 4096, 16
_CT, _CT1, _OUT = 64, 64, 128
_H = _OUT // 2
_CIN = _CT + _CT1 + 3
_R2 = np.float32(0.1 * 0.1)
_EPS = np.float32(1e-5)
_TOT = _B * _N * _K

_QT = 256


def _ballq_body(q_ref, bT_ref, plo_ref, phi_ref, idx_ref):
    b = pl.program_id(0)
    q = q_ref[0]
    sqd = None
    for d in range(3):
        diff = q[:, d : d + 1] - bT_ref[0, d : d + 1, :]
        sq = diff * diff
        sqd = sq if sqd is None else sqd + sq
    maskb = jnp.where(sqd < _R2, np.float32(1), np.float32(0)).astype(
        jnp.bfloat16)
    lo = jnp.dot(maskb, plo_ref[...], preferred_element_type=jnp.float32)
    hi = jnp.dot(maskb, phi_ref[...], preferred_element_type=jnp.float32)
    w = lo.astype(jnp.int32) | (hi.astype(jnp.int32) << 16)

    lane = lax.broadcasted_iota(jnp.int32, (_QT, 128), 1)
    off = b * _N
    firstfix = None
    for j in range(_K):
        wpos = jnp.min(jnp.where(w != 0, lane, 128), axis=1, keepdims=True)
        sel = lane == wpos
        word = jnp.sum(jnp.where(sel, w, 0), axis=1, keepdims=True)
        lsb = word & -word
        f = jnp.abs(lsb.astype(jnp.float32))
        e = (lax.bitcast_convert_type(f, jnp.int32) >> 23) - 127
        cand = wpos * 32 + e
        valid = wpos < 128
        if j == 0:
            firstfix = jnp.where(valid, cand, 0) + off
            idx_ref[0, :, 0:1] = firstfix
        else:
            idx_ref[0, :, j : j + 1] = jnp.where(valid, cand + off, firstfix)
        if j < _K - 1:
            w = jnp.where(sel, w & ~lsb, w)


def _pack_mats():
    n = np.arange(_N)
    w = n // 32
    r = n % 32
    plo = np.zeros((_N, 128), np.float32)
    phi = np.zeros((_N, 128), np.float32)
    sel = r < 16
    plo[n[sel], w[sel]] = 2.0 ** r[sel]
    phi[n[~sel], w[~sel]] = 2.0 ** (r[~sel] - 16)
    return (jnp.asarray(plo, jnp.bfloat16), jnp.asarray(phi, jnp.bfloat16))


def _prep_body(ft1_ref, x1_ref, ft_ref, x_ref, w1t_ref, t_ref, qt_ref):
    w1bt = w1t_ref[_CT : _CT + _CT1, :]
    t = jnp.dot(ft1_ref[...], w1bt, preferred_element_type=jnp.float32,
                precision=lax.Precision.HIGHEST)
    qt = jnp.dot(ft_ref[...], w1t_ref[0:_CT, :], preferred_element_type=jnp.float32,
                 precision=lax.Precision.HIGHEST)
    for d in range(3):
        wrow = w1t_ref[_CT + _CT1 + d : _CT + _CT1 + d + 1, :]
        t = t + x1_ref[:, d : d + 1] * wrow
        qt = qt - x_ref[:, d : d + 1] * wrow
    t_ref[:, 0:_H] = t
    t_ref[:, _H:] = jnp.zeros_like(t)
    qt_ref[...] = qt


def _stats1_body(g_ref, qt_ref, s1_ref, acc_ref):
    step = pl.program_id(0) * _K + pl.program_id(1)
    y = g_ref[0][:, 0:_H] + qt_ref[0]

    @pl.when(step == 0)
    def _():
        acc_ref[...] = jnp.zeros_like(acc_ref)

    acc_ref[0:1, :] += jnp.sum(y, axis=0, keepdims=True)
    acc_ref[1:2, :] += jnp.sum(y * y, axis=0, keepdims=True)

    @pl.when(step == _B * _K - 1)
    def _():
        s1_ref[...] = acc_ref[...]


def _mlp_body(g_ref, qt_ref, s1_ref, w2t_ref, zmax_ref, s2_ref, acc_ref):
    b = pl.program_id(0)
    k = pl.program_id(1)
    step = b * _K + k
    cnt = np.float32(_TOT)
    mu = s1_ref[0:1, :] / cnt
    var = s1_ref[1:2, :] / cnt - mu * mu
    rsig = lax.rsqrt(var + _EPS)
    y = (g_ref[0][:, 0:_H] + qt_ref[0] - mu) * rsig
    y = jnp.maximum(y, 0.0)
    z = jnp.dot(y, w2t_ref[...], preferred_element_type=jnp.float32,
                precision=lax.Precision.DEFAULT)

    @pl.when(step == 0)
    def _():
        acc_ref[...] = jnp.zeros_like(acc_ref)

    acc_ref[0:1, :] += jnp.sum(z, axis=0, keepdims=True)
    acc_ref[1:2, :] += jnp.sum(z * z, axis=0, keepdims=True)

    @pl.when(k == 0)
    def _():
        zmax_ref[0] = z

    @pl.when(k > 0)
    def _():
        zmax_ref[0] = jnp.maximum(zmax_ref[0], z)

    @pl.when(step == _B * _K - 1)
    def _():
        s2_ref[...] = acc_ref[...]


def _final_body(zmax_ref, s2_ref, out_ref):
    cnt = np.float32(_TOT)
    mu = s2_ref[0:1, :] / cnt
    var = s2_ref[1:2, :] / cnt - mu * mu
    rsig = lax.rsqrt(var + _EPS)
    out_ref[0] = jnp.maximum((zmax_ref[0] - mu) * rsig, 0.0)


def _gather_rows(t, idx2d):
    nw = 32
    nrow128 = _TOT // 128
    cpw = nrow128 // nw
    grp = 4
    mesh = plsc.VectorSubcoreMesh(core_axis_name="c", subcore_axis_name="s")

    @functools.partial(
        pl.kernel,
        out_type=jax.ShapeDtypeStruct((_TOT, 128), jnp.float32),
        mesh=mesh,
        scratch_types=[
            pltpu.VMEM((cpw, 128), jnp.int32),
            pltpu.VMEM((grp * 128, 128), jnp.float32),
            pltpu.SemaphoreType.DMA,
        ],
    )
    def gk(idx_hbm, t_hbm, out_hbm, idx_v, rows_v, sem):
        wid = lax.axis_index("s") * 2 + lax.axis_index("c")
        cbase = wid * cpw
        pltpu.sync_copy(idx_hbm.at[pl.ds(cbase, cpw)], idx_v)
        for gi in range(cpw // grp):
            cps = [
                pltpu.async_copy(
                    t_hbm.at[idx_v.at[gi * grp + j]],
                    rows_v.at[pl.ds(j * 128, 128)],
                    sem,
                )
                for j in range(grp)
            ]
            for cp in cps:
                cp.wait()
            pltpu.sync_copy(
                rows_v, out_hbm.at[pl.ds((cbase + gi * grp) * 128, grp * 128)]
            )

    return gk(idx2d, t)


def kernel(xyz_t, feat_t, xyz_t1, feat_t1, W1, W2):
    bT = jnp.transpose(xyz_t1, (0, 2, 1))
    plo, phi = _pack_mats()
    idx = pl.pallas_call(
        _ballq_body,
        grid=(_B, _N // _QT),
        in_specs=[
            pl.BlockSpec((1, _QT, 3), lambda b, i: (b, i, 0)),
            pl.BlockSpec((1, 3, _N), lambda b, i: (b, 0, 0)),
            pl.BlockSpec((_N, 128), lambda b, i: (0, 0)),
            pl.BlockSpec((_N, 128), lambda b, i: (0, 0)),
        ],
        out_specs=pl.BlockSpec((1, _QT, _K), lambda b, i: (b, i, 0)),
        out_shape=jax.ShapeDtypeStruct((_B, _N, _K), jnp.int32),
    )(xyz_t, bT, plo, phi)

    W1T = W1.T
    rt = 1024
    t, qterm = pl.pallas_call(
        _prep_body,
        grid=(_B * _N // rt,),
        in_specs=[
            pl.BlockSpec((rt, _CT1), lambda i: (i, 0)),
            pl.BlockSpec((rt, 3), lambda i: (i, 0)),
            pl.BlockSpec((rt, _CT), lambda i: (i, 0)),
            pl.BlockSpec((rt, 3), lambda i: (i, 0)),
            pl.BlockSpec((_CIN, _H), lambda i: (0, 0)),
        ],
        out_specs=[
            pl.BlockSpec((rt, 128), lambda i: (i, 0)),
            pl.BlockSpec((rt, _H), lambda i: (i, 0)),
        ],
        out_shape=[
            jax.ShapeDtypeStruct((_B * _N, 128), jnp.float32),
            jax.ShapeDtypeStruct((_B * _N, _H), jnp.float32),
        ],
    )(
        feat_t1.reshape(_B * _N, _CT1),
        xyz_t1.reshape(_B * _N, 3),
        feat_t.reshape(_B * _N, _CT),
        xyz_t.reshape(_B * _N, 3),
        W1T,
    )

    idx2d = jnp.transpose(idx, (0, 2, 1)).reshape(_TOT // 128, 128)
    g = _gather_rows(t, idx2d)

    g3 = g.reshape(_B * _K, _N, 128)
    qt3 = qterm.reshape(_B, _N, _H)

    s1 = pl.pallas_call(
        _stats1_body,
        grid=(_B, _K),
        in_specs=[
            pl.BlockSpec((1, _N, 128), lambda b, k: (b * _K + k, 0, 0)),
            pl.BlockSpec((1, _N, _H), lambda b, k: (b, 0, 0)),
        ],
        out_specs=pl.BlockSpec((8, _H), lambda b, k: (0, 0)),
        out_shape=jax.ShapeDtypeStruct((8, _H), jnp.float32),
        scratch_shapes=[pltpu.VMEM((8, _H), jnp.float32)],
    )(g3, qt3)

    zmax, s2 = pl.pallas_call(
        _mlp_body,
        grid=(_B, _K),
        in_specs=[
            pl.BlockSpec((1, _N, 128), lambda b, k: (b * _K + k, 0, 0)),
            pl.BlockSpec((1, _N, _H), lambda b, k: (b, 0, 0)),
            pl.BlockSpec((8, _H), lambda b, k: (0, 0)),
            pl.BlockSpec((_H, _OUT), lambda b, k: (0, 0)),
        ],
        out_specs=[
            pl.BlockSpec((1, _N, _OUT), lambda b, k: (b, 0, 0)),
            pl.BlockSpec((8, _OUT), lambda b, k: (0, 0)),
        ],
        out_shape=[
            jax.ShapeDtypeStruct((_B, _N, _OUT), jnp.float32),
            jax.ShapeDtypeStruct((8, _OUT), jnp.float32),
        ],
        scratch_shapes=[pltpu.VMEM((8, _OUT), jnp.float32)],
    )(g3, qt3, s1, W2.T)

    out = pl.pallas_call(
        _final_body,
        grid=(_B,),
        in_specs=[
            pl.BlockSpec((1, _N, _OUT), lambda b: (b, 0, 0)),
            pl.BlockSpec((8, _OUT), lambda b: (0, 0)),
        ],
        out_specs=pl.BlockSpec((1, _N, _OUT), lambda b: (b, 0, 0)),
        out_shape=jax.ShapeDtypeStruct((_B, _N, _OUT), jnp.float32),
    )(zmax, s2)
    return out


# speedup vs baseline: 29.6335x; 1.2339x over previous
import functools

import jax
import jax.numpy as jnp
import numpy as np
from jax import lax
from jax.experimental import pallas as pl
from jax.experimental.pallas import tpu as pltpu
from jax.experimental.pallas import tpu_sc as plsc

_B, _N, _K = 2, 4096, 16
_CT, _CT1, _OUT = 64, 64, 128
_H = _OUT // 2
_CIN = _CT + _CT1 + 3
_R2 = np.float32(0.1 * 0.1)
_EPS = np.float32(1e-5)
_TOT = _B * _N * _K

_QT = 1024


def _ballq_body(q_ref, bT_ref, plo_ref, phi_ref, idx_ref):
    b = pl.program_id(0)
    q = q_ref[0]
    sqd = None
    for d in range(3):
        diff = q[:, d : d + 1] - bT_ref[0, d : d + 1, :]
        sq = diff * diff
        sqd = sq if sqd is None else sqd + sq
    maskb = jnp.where(sqd < _R2, np.float32(1), np.float32(0)).astype(
        jnp.bfloat16)
    lo = jnp.dot(maskb, plo_ref[...], preferred_element_type=jnp.float32)
    hi = jnp.dot(maskb, phi_ref[...], preferred_element_type=jnp.float32)
    w = lo.astype(jnp.int32) | (hi.astype(jnp.int32) << 16)

    lane = lax.broadcasted_iota(jnp.int32, (_QT, 128), 1)
    off = b * _N
    firstfix = None
    for j in range(_K):
        wpos = jnp.min(jnp.where(w != 0, lane, 128), axis=1, keepdims=True)
        sel = lane == wpos
        word = jnp.sum(jnp.where(sel, w, 0), axis=1, keepdims=True)
        lsb = word & -word
        f = jnp.abs(lsb.astype(jnp.float32))
        e = (lax.bitcast_convert_type(f, jnp.int32) >> 23) - 127
        cand = wpos * 32 + e
        valid = wpos < 128
        if j == 0:
            firstfix = jnp.where(valid, cand, 0) + off
            idx_ref[0, :, 0:1] = firstfix
        else:
            idx_ref[0, :, j : j + 1] = jnp.where(valid, cand + off, firstfix)
        if j < _K - 1:
            w = jnp.where(sel, w & ~lsb, w)


def _pack_mats():
    n = np.arange(_N)
    w = n // 32
    r = n % 32
    plo = np.zeros((_N, 128), np.float32)
    phi = np.zeros((_N, 128), np.float32)
    sel = r < 16
    plo[n[sel], w[sel]] = 2.0 ** r[sel]
    phi[n[~sel], w[~sel]] = 2.0 ** (r[~sel] - 16)
    return (jnp.asarray(plo, jnp.bfloat16), jnp.asarray(phi, jnp.bfloat16))


def _prep_body(ft1_ref, x1_ref, ft_ref, x_ref, w1t_ref, t_ref, qt_ref):
    w1bt = w1t_ref[_CT : _CT + _CT1, :]
    t = jnp.dot(ft1_ref[...], w1bt, preferred_element_type=jnp.float32,
                precision=lax.Precision.HIGHEST)
    qt = jnp.dot(ft_ref[...], w1t_ref[0:_CT, :], preferred_element_type=jnp.float32,
                 precision=lax.Precision.HIGHEST)
    for d in range(3):
        wrow = w1t_ref[_CT + _CT1 + d : _CT + _CT1 + d + 1, :]
        t = t + x1_ref[:, d : d + 1] * wrow
        qt = qt - x_ref[:, d : d + 1] * wrow
    t_ref[:, 0:_H] = t
    t_ref[:, _H:] = jnp.zeros_like(t)
    qt_ref[...] = qt


def _stats1_body(g_ref, qt_ref, s1_ref, acc_ref):
    step = pl.program_id(0) * _K + pl.program_id(1)
    y = g_ref[0][:, 0:_H] + qt_ref[0]

    @pl.when(step == 0)
    def _():
        acc_ref[...] = jnp.zeros_like(acc_ref)

    acc_ref[0:1, :] += jnp.sum(y, axis=0, keepdims=True)
    acc_ref[1:2, :] += jnp.sum(y * y, axis=0, keepdims=True)

    @pl.when(step == _B * _K - 1)
    def _():
        s1_ref[...] = acc_ref[...]


def _mlp_body(g_ref, qt_ref, s1_ref, w2t_ref, zmax_ref, s2_ref, acc_ref):
    b = pl.program_id(0)
    k = pl.program_id(1)
    step = b * _K + k
    cnt = np.float32(_TOT)
    mu = s1_ref[0:1, :] / cnt
    var = s1_ref[1:2, :] / cnt - mu * mu
    rsig = lax.rsqrt(var + _EPS)
    y = (g_ref[0][:, 0:_H] + qt_ref[0] - mu) * rsig
    y = jnp.maximum(y, 0.0)
    z = jnp.dot(y, w2t_ref[...], preferred_element_type=jnp.float32,
                precision=lax.Precision.DEFAULT)

    @pl.when(step == 0)
    def _():
        acc_ref[...] = jnp.zeros_like(acc_ref)

    acc_ref[0:1, :] += jnp.sum(z, axis=0, keepdims=True)
    acc_ref[1:2, :] += jnp.sum(z * z, axis=0, keepdims=True)

    @pl.when(k == 0)
    def _():
        zmax_ref[0] = z

    @pl.when(k > 0)
    def _():
        zmax_ref[0] = jnp.maximum(zmax_ref[0], z)

    @pl.when(step == _B * _K - 1)
    def _():
        s2_ref[...] = acc_ref[...]


def _final_body(zmax_ref, s2_ref, out_ref):
    cnt = np.float32(_TOT)
    mu = s2_ref[0:1, :] / cnt
    var = s2_ref[1:2, :] / cnt - mu * mu
    rsig = lax.rsqrt(var + _EPS)
    out_ref[0] = jnp.maximum((zmax_ref[0] - mu) * rsig, 0.0)


def _gather_rows(t, idx2d):
    nw = 32
    nrow128 = _TOT // 128
    cpw = nrow128 // nw
    grp = 4
    mesh = plsc.VectorSubcoreMesh(core_axis_name="c", subcore_axis_name="s")

    @functools.partial(
        pl.kernel,
        out_type=jax.ShapeDtypeStruct((_TOT, 128), jnp.float32),
        mesh=mesh,
        scratch_types=[
            pltpu.VMEM((cpw, 128), jnp.int32),
            pltpu.VMEM((grp * 128, 128), jnp.float32),
            pltpu.SemaphoreType.DMA,
        ],
    )
    def gk(idx_hbm, t_hbm, out_hbm, idx_v, rows_v, sem):
        wid = lax.axis_index("s") * 2 + lax.axis_index("c")
        cbase = wid * cpw
        pltpu.sync_copy(idx_hbm.at[pl.ds(cbase, cpw)], idx_v)
        for gi in range(cpw // grp):
            cps = [
                pltpu.async_copy(
                    t_hbm.at[idx_v.at[gi * grp + j]],
                    rows_v.at[pl.ds(j * 128, 128)],
                    sem,
                )
                for j in range(grp)
            ]
            for cp in cps:
                cp.wait()
            pltpu.sync_copy(
                rows_v, out_hbm.at[pl.ds((cbase + gi * grp) * 128, grp * 128)]
            )

    return gk(idx2d, t)


def kernel(xyz_t, feat_t, xyz_t1, feat_t1, W1, W2):
    bT = jnp.transpose(xyz_t1, (0, 2, 1))
    plo, phi = _pack_mats()
    idx = pl.pallas_call(
        _ballq_body,
        grid=(_B, _N // _QT),
        in_specs=[
            pl.BlockSpec((1, _QT, 3), lambda b, i: (b, i, 0)),
            pl.BlockSpec((1, 3, _N), lambda b, i: (b, 0, 0)),
            pl.BlockSpec((_N, 128), lambda b, i: (0, 0)),
            pl.BlockSpec((_N, 128), lambda b, i: (0, 0)),
        ],
        out_specs=pl.BlockSpec((1, _QT, _K), lambda b, i: (b, i, 0)),
        out_shape=jax.ShapeDtypeStruct((_B, _N, _K), jnp.int32),
    )(xyz_t, bT, plo, phi)

    W1T = W1.T
    rt = 1024
    t, qterm = pl.pallas_call(
        _prep_body,
        grid=(_B * _N // rt,),
        in_specs=[
            pl.BlockSpec((rt, _CT1), lambda i: (i, 0)),
            pl.BlockSpec((rt, 3), lambda i: (i, 0)),
            pl.BlockSpec((rt, _CT), lambda i: (i, 0)),
            pl.BlockSpec((rt, 3), lambda i: (i, 0)),
            pl.BlockSpec((_CIN, _H), lambda i: (0, 0)),
        ],
        out_specs=[
            pl.BlockSpec((rt, 128), lambda i: (i, 0)),
            pl.BlockSpec((rt, _H), lambda i: (i, 0)),
        ],
        out_shape=[
            jax.ShapeDtypeStruct((_B * _N, 128), jnp.float32),
            jax.ShapeDtypeStruct((_B * _N, _H), jnp.float32),
        ],
    )(
        feat_t1.reshape(_B * _N, _CT1),
        xyz_t1.reshape(_B * _N, 3),
        feat_t.reshape(_B * _N, _CT),
        xyz_t.reshape(_B * _N, 3),
        W1T,
    )

    idx2d = jnp.transpose(idx, (0, 2, 1)).reshape(_TOT // 128, 128)
    g = _gather_rows(t, idx2d)

    g3 = g.reshape(_B * _K, _N, 128)
    qt3 = qterm.reshape(_B, _N, _H)

    s1 = pl.pallas_call(
        _stats1_body,
        grid=(_B, _K),
        in_specs=[
            pl.BlockSpec((1, _N, 128), lambda b, k: (b * _K + k, 0, 0)),
            pl.BlockSpec((1, _N, _H), lambda b, k: (b, 0, 0)),
        ],
        out_specs=pl.BlockSpec((8, _H), lambda b, k: (0, 0)),
        out_shape=jax.ShapeDtypeStruct((8, _H), jnp.float32),
        scratch_shapes=[pltpu.VMEM((8, _H), jnp.float32)],
    )(g3, qt3)

    zmax, s2 = pl.pallas_call(
        _mlp_body,
        grid=(_B, _K),
        in_specs=[
            pl.BlockSpec((1, _N, 128), lambda b, k: (b * _K + k, 0, 0)),
            pl.BlockSpec((1, _N, _H), lambda b, k: (b, 0, 0)),
            pl.BlockSpec((8, _H), lambda b, k: (0, 0)),
            pl.BlockSpec((_H, _OUT), lambda b, k: (0, 0)),
        ],
        out_specs=[
            pl.BlockSpec((1, _N, _OUT), lambda b, k: (b, 0, 0)),
            pl.BlockSpec((8, _OUT), lambda b, k: (0, 0)),
        ],
        out_shape=[
            jax.ShapeDtypeStruct((_B, _N, _OUT), jnp.float32),
            jax.ShapeDtypeStruct((8, _OUT), jnp.float32),
        ],
        scratch_shapes=[pltpu.VMEM((8, _OUT), jnp.float32)],
    )(g3, qt3, s1, W2.T)

    out = pl.pallas_call(
        _final_body,
        grid=(_B,),
        in_specs=[
            pl.BlockSpec((1, _N, _OUT), lambda b: (b, 0, 0)),
            pl.BlockSpec((8, _OUT), lambda b: (0, 0)),
        ],
        out_specs=pl.BlockSpec((1, _N, _OUT), lambda b: (b, 0, 0)),
        out_shape=jax.ShapeDtypeStruct((_B, _N, _OUT), jnp.float32),
    )(zmax, s2)
    return out
